# Initial kernel scaffold; baseline (speedup 1.0000x reference)
import jax
import jax.numpy as jnp
from jax.experimental import pallas as pl


def kernel(img0, img1):
    raise NotImplementedError("write your pallas kernel here")


# speedup vs baseline: 67.6437x; 67.6437x over previous
import functools

import jax
import jax.numpy as jnp
from jax import lax
from jax.experimental import pallas as pl
from jax.experimental.pallas import tpu as pltpu
from jax.experimental.pallas import tpu_sc as plsc

NBIN = 256
L = 16
NW = 32
N_ELEM = 4096 * 4096
EPW = N_ELEM // NW
CH = 32768
NCH = EPW // CH
CHV = CH // L
UNROLL = 8
HSTRIDE = 1024
HWORDS = L * HSTRIDE


def _min_body(x_ref, o_ref):
    m = jnp.min(x_ref[...])

    @pl.when(pl.program_id(0) == 0)
    def _():
        o_ref[0, 0] = m

    @pl.when(pl.program_id(0) > 0)
    def _():
        o_ref[0, 0] = jnp.minimum(o_ref[0, 0], m)


def _tc_min(img0):
    return pl.pallas_call(
        _min_body,
        grid=(16,),
        in_specs=[pl.BlockSpec((256, 4096), lambda i: (i, 0))],
        out_specs=pl.BlockSpec(memory_space=pltpu.SMEM),
        out_shape=jax.ShapeDtypeStruct((1, 1), jnp.float32),
    )(img0)


def _sc_hist_body(img0_ref, img1_ref, min_ref, out_ref,
                  minbuf, hist, buf0, buf1, sem0, sem1):
    cid = lax.axis_index("c")
    sid = lax.axis_index("s")
    wid = sid * 2 + cid
    base = wid * EPW

    zeros = jnp.zeros((L,), jnp.float32)

    def _zero(i, carry):
        hist[pl.ds(i * L, L)] = zeros
        return carry

    lax.fori_loop(0, HWORDS // L, _zero, 0)

    pltpu.sync_copy(min_ref, minbuf)
    hmin = minbuf[...]
    inv_dh = (NBIN - 1.0) / (0.0 - hmin)
    lane_base = lax.broadcasted_iota(jnp.int32, (L,), 0) * HSTRIDE

    bufs = (buf0, buf1)
    sems = (sem0, sem1)

    def _issue(c, b):
        @pl.when(c < NCH)
        def _():
            pltpu.async_copy(
                img0_ref.at[pl.ds(base + c * CH, CH)], bufs[b], sems[b])

        @pl.when(jnp.logical_and(c >= NCH, c < 2 * NCH))
        def _():
            pltpu.async_copy(
                img1_ref.at[pl.ds(base + (c - NCH) * CH, CH)], bufs[b], sems[b])

    _issue(jnp.int32(0), 0)

    def _outer(c2, carry):
        for b in range(2):
            c = c2 * 2 + b
            _issue(c + 1, 1 - b)
            pltpu.make_async_copy(
                img0_ref.at[pl.ds(0, CH)], bufs[b], sems[b]).wait()
            ho = jnp.where(c < NCH, 0, 512).astype(jnp.int32)
            laneho = lane_base + ho

            def _inner(j, carry2, _b=b, _laneho=laneho):
                for u in range(UNROLL):
                    x = bufs[_b][pl.ds((j * UNROLL + u) * L, L)]
                    t = (x - hmin) * inv_dh
                    ti = t.astype(jnp.int32)
                    f = t - ti.astype(jnp.float32)
                    keep = jnp.logical_and(x >= hmin, x <= 0.0)
                    tic = jnp.clip(ti, 0, NBIN - 1)
                    fl0 = _laneho + tic
                    plsc.addupdate_scatter(hist, [fl0], 1.0 - f, mask=keep)
                    plsc.addupdate_scatter(hist, [fl0 + 1], f, mask=keep)
                return carry2

            lax.fori_loop(0, CHV // UNROLL, _inner, 0)
        return carry

    lax.fori_loop(0, NCH, _outer, 0)

    pltpu.sync_copy(hist, out_ref.at[pl.ds(wid * HWORDS, HWORDS)])


def _sc_hist(img0_flat, img1_flat, minv):
    mesh = plsc.VectorSubcoreMesh(core_axis_name="c", subcore_axis_name="s")
    return pl.kernel(
        _sc_hist_body,
        out_type=jax.ShapeDtypeStruct((NW * HWORDS,), jnp.float32),
        mesh=mesh,
        scratch_types=[
            pltpu.VMEM((L,), jnp.float32),
            pltpu.VMEM((HWORDS,), jnp.float32),
            pltpu.VMEM((CH,), jnp.float32),
            pltpu.VMEM((CH,), jnp.float32),
            pltpu.SemaphoreType.DMA,
            pltpu.SemaphoreType.DMA,
        ],
        compiler_params=pltpu.CompilerParams(needs_layout_passes=False),
    )(img0_flat, img1_flat, minv)


def _kl_body(p_ref, o_ref):
    s = jnp.sum(p_ref[...], axis=0, keepdims=True)
    h0 = s[:, 0:NBIN]
    h1 = s[:, 512:512 + NBIN]
    eps = 1e-10
    H0 = (h0 + eps) / (jnp.sum(h0) + eps)
    H1 = (h1 + eps) / (jnp.sum(h1) + eps)
    inp = jnp.log((H1 + eps) / H1)
    tgt = jnp.log((H1 + eps) / H0)
    o_ref[0, 0] = jnp.mean(jnp.exp(tgt) * (tgt - inp))


def _tc_kl(partials):
    return pl.pallas_call(
        _kl_body,
        in_specs=[pl.BlockSpec((NW * L, HSTRIDE), lambda: (0, 0))],
        out_specs=pl.BlockSpec(memory_space=pltpu.SMEM),
        out_shape=jax.ShapeDtypeStruct((1, 1), jnp.float32),
    )(partials)


@jax.jit
def kernel(img0, img1):
    min0 = _tc_min(img0)
    minv = jnp.broadcast_to(min0.reshape(()), (L,))
    partials = _sc_hist(img0.reshape(-1), img1.reshape(-1), minv)
    loss = _tc_kl(partials.reshape(NW * L, HSTRIDE))
    return loss[0, 0]


# speedup vs baseline: 203.9721x; 3.0154x over previous
import functools

import jax
import jax.numpy as jnp
from jax import lax
from jax.experimental import pallas as pl
from jax.experimental.pallas import tpu as pltpu
from jax.experimental.pallas import tpu_sc as plsc

NBIN = 256
L = 16
NW = 32
N_ELEM = 4096 * 4096
EPW = N_ELEM // NW
CH = 32768
NCH = EPW // CH
CHV = CH // L
UNROLL = 8
HSTRIDE = 1024
HWORDS = L * HSTRIDE
HO1 = 512


def _min_body(x_ref, o_ref):
    m = jnp.min(x_ref[...])

    @pl.when(pl.program_id(0) == 0)
    def _():
        o_ref[0, 0] = m

    @pl.when(pl.program_id(0) > 0)
    def _():
        o_ref[0, 0] = jnp.minimum(o_ref[0, 0], m)


def _tc_min(img0):
    return pl.pallas_call(
        _min_body,
        grid=(16,),
        in_specs=[pl.BlockSpec((256, 4096), lambda i: (i, 0))],
        out_specs=pl.BlockSpec(memory_space=pltpu.SMEM),
        out_shape=jax.ShapeDtypeStruct((1, 1), jnp.float32),
    )(img0)


def _sc_hist_body(img0_ref, img1_ref, min_ref, out_ref,
                  minbuf, hist, buf0, buf1, sem0, sem1):
    cid = lax.axis_index("c")
    sid = lax.axis_index("s")
    wid = sid * 2 + cid
    base = wid * EPW

    zeros = jnp.zeros((L,), jnp.float32)

    @plsc.parallel_loop(0, HWORDS // L, unroll=8)
    def _zero(i):
        hist[pl.ds(i * L, L)] = zeros

    pltpu.sync_copy(min_ref, minbuf)
    hmin = minbuf[...]
    inv_dh = (NBIN - 1.0) / (0.0 - hmin)
    lane_skew = lax.broadcasted_iota(jnp.int32, (L,), 0) * (HSTRIDE + 1)

    bufs = (buf0, buf1)
    sems = (sem0, sem1)

    def _phase(img_ref, laneho, check_lo):
        def _issue(c, b):
            @pl.when(c < NCH)
            def _():
                pltpu.async_copy(
                    img_ref.at[pl.ds(base + c * CH, CH)], bufs[b], sems[b])

        _issue(jnp.int32(0), 0)

        def _outer(c2, carry):
            for b in range(2):
                c = c2 * 2 + b
                _issue(c + 1, 1 - b)
                pltpu.make_async_copy(
                    img_ref.at[pl.ds(0, CH)], bufs[b], sems[b]).wait()

                @plsc.parallel_loop(0, CHV, unroll=UNROLL)
                def _inner(j, _b=b):
                    x = bufs[_b][pl.ds(j * L, L)]
                    t = x * inv_dh + (NBIN - 1.0)
                    ti = t.astype(jnp.int32)
                    f = t - ti.astype(jnp.float32)
                    if check_lo:
                        keep = jnp.logical_and(t >= 0.0, t <= NBIN - 1.0)
                    else:
                        keep = t <= NBIN - 1.0
                    tic = jnp.clip(ti, 0, NBIN - 1)
                    fl0 = laneho + tic
                    plsc.addupdate_scatter(hist, [fl0], 1.0 - f, mask=keep)
                    plsc.addupdate_scatter(hist, [fl0 + 1], f, mask=keep)
            return carry

        lax.fori_loop(0, NCH // 2, _outer, 0)

    _phase(img0_ref, lane_skew, False)
    _phase(img1_ref, lane_skew + HO1, True)

    pltpu.sync_copy(hist, out_ref.at[pl.ds(wid * HWORDS, HWORDS)])


def _sc_hist(img0_flat, img1_flat, minv):
    mesh = plsc.VectorSubcoreMesh(core_axis_name="c", subcore_axis_name="s")
    return pl.kernel(
        _sc_hist_body,
        out_type=jax.ShapeDtypeStruct((NW * HWORDS,), jnp.float32),
        mesh=mesh,
        scratch_types=[
            pltpu.VMEM((L,), jnp.float32),
            pltpu.VMEM((HWORDS,), jnp.float32),
            pltpu.VMEM((CH,), jnp.float32),
            pltpu.VMEM((CH,), jnp.float32),
            pltpu.SemaphoreType.DMA,
            pltpu.SemaphoreType.DMA,
        ],
        compiler_params=pltpu.CompilerParams(needs_layout_passes=False),
    )(img0_flat, img1_flat, minv)


def _kl_body(p_ref, o_ref):
    s = jnp.sum(p_ref[...], axis=0)
    h0 = jnp.zeros((1, NBIN), jnp.float32)
    h1 = jnp.zeros((1, NBIN), jnp.float32)
    for l in range(L):
        h0 = h0 + s[l:l + 1, l:l + NBIN]
        h1 = h1 + s[l:l + 1, HO1 + l:HO1 + l + NBIN]
    eps = 1e-10
    H0 = (h0 + eps) / (jnp.sum(h0) + eps)
    H1 = (h1 + eps) / (jnp.sum(h1) + eps)
    inp = jnp.log((H1 + eps) / H1)
    tgt = jnp.log((H1 + eps) / H0)
    o_ref[0, 0] = jnp.mean(jnp.exp(tgt) * (tgt - inp))


def _tc_kl(partials):
    return pl.pallas_call(
        _kl_body,
        in_specs=[pl.BlockSpec((NW, L, HSTRIDE), lambda: (0, 0, 0))],
        out_specs=pl.BlockSpec(memory_space=pltpu.SMEM),
        out_shape=jax.ShapeDtypeStruct((1, 1), jnp.float32),
    )(partials)


@jax.jit
def kernel(img0, img1):
    min0 = _tc_min(img0)
    minv = jnp.broadcast_to(min0.reshape(()), (L,))
    partials = _sc_hist(img0.reshape(-1), img1.reshape(-1), minv)
    loss = _tc_kl(partials.reshape(NW, L, HSTRIDE))
    return loss[0, 0]


# speedup vs baseline: 262.8142x; 1.2885x over previous
import functools

import jax
import jax.numpy as jnp
from jax import lax
from jax.experimental import pallas as pl
from jax.experimental.pallas import tpu as pltpu
from jax.experimental.pallas import tpu_sc as plsc

NBIN = 256
L = 16
NW = 32
N_ELEM = 4096 * 4096
EPW = N_ELEM // NW
CH = 32768
NCH = EPW // CH
CHV = CH // L
UNROLL = 8
HSTRIDE = 1024
HWORDS = L * HSTRIDE
HO1 = 512


def _min_body(x_ref, o_ref):
    m = jnp.min(x_ref[...])

    @pl.when(pl.program_id(0) == 0)
    def _():
        o_ref[0, 0] = m

    @pl.when(pl.program_id(0) > 0)
    def _():
        o_ref[0, 0] = jnp.minimum(o_ref[0, 0], m)


def _tc_min(img0):
    return pl.pallas_call(
        _min_body,
        grid=(16,),
        in_specs=[pl.BlockSpec((256, 4096), lambda i: (i, 0))],
        out_specs=pl.BlockSpec(memory_space=pltpu.SMEM),
        out_shape=jax.ShapeDtypeStruct((1, 1), jnp.float32),
    )(img0)


ROWS = 4096
RPW = ROWS // NW
RCH = 8
RV = 4096 // L


def _sc_hist_body(img0_ref, img1_ref, min_ref, out_ref,
                  minbuf, hist, buf0, buf1, sem0, sem1):
    cid = lax.axis_index("c")
    sid = lax.axis_index("s")
    wid = sid * 2 + cid
    rowbase = wid * RPW

    zeros = jnp.zeros((L,), jnp.float32)

    @plsc.parallel_loop(0, HWORDS // L, unroll=8)
    def _zero(i):
        hist[pl.ds(i * L, L)] = zeros

    pltpu.sync_copy(min_ref, minbuf)
    hmin = minbuf[pl.ds(0, L)]
    inv_dh = (NBIN - 1.0) / (0.0 - hmin)
    lane_skew = lax.broadcasted_iota(jnp.int32, (L,), 0) * (HSTRIDE + 1)

    bufs = (buf0, buf1)
    sems = (sem0, sem1)

    def _phase(img_ref, laneho, check_lo):
        def _issue(c, b):
            @pl.when(c < NCH)
            def _():
                pltpu.async_copy(
                    img_ref.at[pl.ds(rowbase + c * RCH, RCH)],
                    bufs[b], sems[b])

        _issue(jnp.int32(0), 0)

        def _outer(c2, carry):
            for b in range(2):
                c = c2 * 2 + b
                _issue(c + 1, 1 - b)
                pltpu.make_async_copy(
                    img_ref.at[pl.ds(0, RCH)], bufs[b], sems[b]).wait()

                for r in range(RCH):
                    @plsc.parallel_loop(0, RV, unroll=UNROLL)
                    def _inner(j, _b=b, _r=r):
                        x = bufs[_b][_r, pl.ds(j * L, L)]
                        t = x * inv_dh + (NBIN - 1.0)
                        ti = t.astype(jnp.int32)
                        f = t - ti.astype(jnp.float32)
                        if check_lo:
                            keep = jnp.logical_and(t >= 0.0, t <= NBIN - 1.0)
                        else:
                            keep = t <= NBIN - 1.0
                        tic = jnp.clip(ti, 0, NBIN - 1)
                        fl0 = laneho + tic
                        plsc.addupdate_scatter(hist, [fl0], 1.0 - f, mask=keep)
                        plsc.addupdate_scatter(hist, [fl0 + 1], f, mask=keep)
            return carry

        lax.fori_loop(0, NCH // 2, _outer, 0)

    _phase(img0_ref, lane_skew, False)
    _phase(img1_ref, lane_skew + HO1, True)

    pltpu.sync_copy(hist, out_ref.at[pl.ds(wid * HWORDS, HWORDS)])


def _sc_hist(img0, img1, minv):
    mesh = plsc.VectorSubcoreMesh(core_axis_name="c", subcore_axis_name="s")
    return pl.kernel(
        _sc_hist_body,
        out_type=jax.ShapeDtypeStruct((NW * HWORDS,), jnp.float32),
        mesh=mesh,
        scratch_types=[
            pltpu.VMEM((128,), jnp.float32),
            pltpu.VMEM((HWORDS,), jnp.float32),
            pltpu.VMEM((RCH, 4096), jnp.float32),
            pltpu.VMEM((RCH, 4096), jnp.float32),
            pltpu.SemaphoreType.DMA,
            pltpu.SemaphoreType.DMA,
        ],
        compiler_params=pltpu.CompilerParams(
            needs_layout_passes=False, use_tc_tiling_on_sc=True),
    )(img0, img1, minv)


def _kl_body(p_ref, o_ref):
    s = jnp.sum(p_ref[...], axis=0)
    h0 = jnp.zeros((1, NBIN), jnp.float32)
    h1 = jnp.zeros((1, NBIN), jnp.float32)
    for l in range(L):
        h0 = h0 + s[l:l + 1, l:l + NBIN]
        h1 = h1 + s[l:l + 1, HO1 + l:HO1 + l + NBIN]
    eps = 1e-10
    H0 = (h0 + eps) / (jnp.sum(h0) + eps)
    H1 = (h1 + eps) / (jnp.sum(h1) + eps)
    inp = jnp.log((H1 + eps) / H1)
    tgt = jnp.log((H1 + eps) / H0)
    o_ref[0, 0] = jnp.mean(jnp.exp(tgt) * (tgt - inp))


def _tc_kl(partials):
    return pl.pallas_call(
        _kl_body,
        in_specs=[pl.BlockSpec((NW, L, HSTRIDE), lambda: (0, 0, 0))],
        out_specs=pl.BlockSpec(memory_space=pltpu.SMEM),
        out_shape=jax.ShapeDtypeStruct((1, 1), jnp.float32),
    )(partials)


@jax.jit
def kernel(img0, img1):
    min0 = _tc_min(img0)
    minv = jnp.broadcast_to(min0.reshape(()), (128,))
    partials = _sc_hist(img0, img1, minv)
    loss = _tc_kl(partials.reshape(NW, L, HSTRIDE))
    return loss[0, 0]


# speedup vs baseline: 302.8302x; 1.1523x over previous
import jax
import jax.numpy as jnp
from jax import lax
from jax.experimental import pallas as pl
from jax.experimental.pallas import tpu as pltpu
from jax.experimental.pallas import tpu_sc as plsc

NBIN = 256
L = 16
NW = 32
ROWS = 4096
RPW = ROWS // NW
RCH = 8
NCH = RPW // RCH
RV = 4096 // L
UNROLL = 8
HSTRIDE = 1041
HWORDS = L * HSTRIDE
C0_OFF = 0
G_OFF = 260
C1_OFF = 520


def _min_body(x_ref, o_ref):
    m = jnp.min(x_ref[...])

    @pl.when(pl.program_id(0) == 0)
    def _():
        o_ref[0, 0] = m

    @pl.when(pl.program_id(0) > 0)
    def _():
        o_ref[0, 0] = jnp.minimum(o_ref[0, 0], m)


def _tc_min(img0):
    return pl.pallas_call(
        _min_body,
        grid=(16,),
        in_specs=[pl.BlockSpec((256, 4096), lambda i: (i, 0))],
        out_specs=pl.BlockSpec(memory_space=pltpu.SMEM),
        out_shape=jax.ShapeDtypeStruct((1, 1), jnp.float32),
    )(img0)


def _sc_hist_body(img0_ref, img1_ref, min_ref, out_ref,
                  minbuf, hist, buf0, buf1, sem0, sem1):
    cid = lax.axis_index("c")
    sid = lax.axis_index("s")
    wid = sid * 2 + cid
    rowbase = wid * RPW

    zeros = jnp.zeros((L,), jnp.float32)
    ones = jnp.full((L,), 1.0, jnp.float32)

    @plsc.parallel_loop(0, HWORDS // L, unroll=8)
    def _zero(i):
        hist[pl.ds(i * L, L)] = zeros

    pltpu.sync_copy(min_ref, minbuf)
    hmin = minbuf[pl.ds(0, L)]
    inv_dh = (NBIN - 1.0) / (0.0 - hmin)
    lane_skew = lax.broadcasted_iota(jnp.int32, (L,), 0) * HSTRIDE

    bufs = (buf0, buf1)
    sems = (sem0, sem1)

    def _phase(img_ref, lanec):
        def _issue(c, b):
            @pl.when(c < NCH)
            def _():
                pltpu.async_copy(
                    img_ref.at[pl.ds(rowbase + c * RCH, RCH)],
                    bufs[b], sems[b])

        _issue(jnp.int32(0), 0)

        def _outer(c2, carry):
            for b in range(2):
                c = c2 * 2 + b
                _issue(c + 1, 1 - b)
                pltpu.make_async_copy(
                    img_ref.at[pl.ds(0, RCH)], bufs[b], sems[b]).wait()

                for r in range(RCH):
                    @plsc.parallel_loop(0, RV, unroll=UNROLL)
                    def _inner(j, _b=b, _r=r):
                        x = bufs[_b][_r, pl.ds(j * L, L)]
                        t = x * inv_dh + (NBIN - 1.0)
                        tcl = jnp.clip(t, 0.0, NBIN - 1.0)
                        keep = t == tcl
                        ti = tcl.astype(jnp.int32)
                        f = tcl - ti.astype(jnp.float32)
                        flc = lanec + ti
                        plsc.addupdate_scatter(hist, [flc], ones, mask=keep)
                        plsc.addupdate_scatter(
                            hist, [flc + G_OFF], f, mask=keep)
            return carry

        lax.fori_loop(0, NCH // 2, _outer, 0)

    _phase(img0_ref, lane_skew + C0_OFF)
    _phase(img1_ref, lane_skew + C1_OFF)

    pltpu.sync_copy(hist, out_ref.at[pl.ds(wid * HWORDS, HWORDS)])


def _sc_hist(img0, img1, minv):
    mesh = plsc.VectorSubcoreMesh(core_axis_name="c", subcore_axis_name="s")
    return pl.kernel(
        _sc_hist_body,
        out_type=jax.ShapeDtypeStruct((NW * HWORDS,), jnp.float32),
        mesh=mesh,
        scratch_types=[
            pltpu.VMEM((128,), jnp.float32),
            pltpu.VMEM((HWORDS,), jnp.float32),
            pltpu.VMEM((RCH, 4096), jnp.float32),
            pltpu.VMEM((RCH, 4096), jnp.float32),
            pltpu.SemaphoreType.DMA,
            pltpu.SemaphoreType.DMA,
        ],
        compiler_params=pltpu.CompilerParams(
            needs_layout_passes=False, use_tc_tiling_on_sc=True),
    )(img0, img1, minv)


def _kl_body(p_ref, o_ref):
    s = jnp.sum(p_ref[...], axis=0, keepdims=True)
    h0 = (s[:, C0_OFF:C0_OFF + NBIN]
          - s[:, C0_OFF + G_OFF:C0_OFF + G_OFF + NBIN]
          + s[:, C0_OFF + G_OFF - 1:C0_OFF + G_OFF - 1 + NBIN])
    h1 = (s[:, C1_OFF:C1_OFF + NBIN]
          - s[:, C1_OFF + G_OFF:C1_OFF + G_OFF + NBIN]
          + s[:, C1_OFF + G_OFF - 1:C1_OFF + G_OFF - 1 + NBIN])
    eps = 1e-10
    H0 = (h0 + eps) / (jnp.sum(h0) + eps)
    H1 = (h1 + eps) / (jnp.sum(h1) + eps)
    inp = jnp.log((H1 + eps) / H1)
    tgt = jnp.log((H1 + eps) / H0)
    o_ref[0, 0] = jnp.mean(jnp.exp(tgt) * (tgt - inp))


def _tc_kl(partials):
    return pl.pallas_call(
        _kl_body,
        in_specs=[pl.BlockSpec((NW * L, HSTRIDE), lambda: (0, 0))],
        out_specs=pl.BlockSpec(memory_space=pltpu.SMEM),
        out_shape=jax.ShapeDtypeStruct((1, 1), jnp.float32),
    )(partials)


@jax.jit
def kernel(img0, img1):
    min0 = _tc_min(img0)
    minv = jnp.broadcast_to(min0.reshape(()), (128,))
    partials = _sc_hist(img0, img1, minv)
    loss = _tc_kl(partials.reshape(NW * L, HSTRIDE))
    return loss[0, 0]
